# baseline (speedup 1.0000x reference)
import functools

import jax
import jax.numpy as jnp
from jax import lax
from jax.experimental import pallas as pl
from jax.experimental.pallas import tpu as pltpu
from jax.experimental.pallas import tpu_sc as plsc

N = 10000
E = 320000
D = 128

NC = 2
NS = 16
NW = NC * NS

K = 64
NCHUNK = 162
EPT = NCHUNK * K
E_PAD = NW * EPT

SUPER = 27
NSUPER = NCHUNK // SUPER
NTRIP = SUPER // 3

N_ACC = 10240
ROWS_PER_TILE = N_ACC // NS
ZCHUNK = 64


def _sc_body(x_hbm, src_hbm, dst_hbm, w_hbm, out_hbm,
             src0_v, src1_v, dst0_v, dst1_v, w0_v, w1_v,
             rows0_v, rows1_v, rows2_v, acc_sh,
             sem_r0, sem_r1, sem_r2, sem_s0, sem_s1, sem_s2,
             sem_i0, sem_i1):
    cid = lax.axis_index("c")
    sid = lax.axis_index("s")
    wid = sid * NC + cid
    rows = (rows0_v, rows1_v, rows2_v)
    sem_r = (sem_r0, sem_r1, sem_r2)
    sem_s = (sem_s0, sem_s1, sem_s2)

    with jax.named_scope("idx_slab"):
        pltpu.async_copy(src_hbm.at[wid, 0], src0_v, sem_i0)
        pltpu.async_copy(dst_hbm.at[wid, 0], dst0_v, sem_i0)
        pltpu.async_copy(w_hbm.at[wid, 0], w0_v, sem_i0)

    with jax.named_scope("zero_acc"):
        def _zero_body(i, carry):
            for c in range(D // 16):
                rows0_v[i, pl.ds(c * 16, 16)] = jnp.zeros((16,), jnp.float32)
            return carry
        lax.fori_loop(0, ZCHUNK, _zero_body, 0)
        for r in range(ROWS_PER_TILE // ZCHUNK):
            pltpu.sync_copy(rows0_v,
                            acc_sh.at[pl.ds(sid * ROWS_PER_TILE + r * ZCHUNK,
                                            ZCHUNK)])
        plsc.subcore_barrier()

    def _gather(srcb, lp, b):
        pltpu.async_copy(x_hbm.at[srcb.at[lp, 0]], rows[b], sem_r[b])

    def _wait_gather(b):
        pltpu.make_async_copy(x_hbm.at[src0_v.at[0, 0]], rows[b],
                              sem_r[b]).wait()

    def _scale(wb, lp, rows_v):
        def _scale_body(g, c2):
            wv = wb[lp, 0, pl.ds(g * 16, 16)]
            for i in range(16):
                wi = wv[i]
                e = g * 16 + i
                for c in range(D // 16):
                    rows_v[e, pl.ds(c * 16, 16)] = (
                        rows_v[e, pl.ds(c * 16, 16)] * wi)
            return c2
        lax.fori_loop(0, K // 16, _scale_body, 0)

    def _scatter(dstb, lp, b):
        pltpu.async_copy(rows[b], acc_sh.at[dstb.at[lp, 0]], sem_s[b],
                         add=True)

    def _wait_scatter(b):
        pltpu.make_async_copy(rows[b], acc_sh.at[dst0_v.at[0, 0]],
                              sem_s[b]).wait()

    def _wait_idx(srcb, dstb, wb, s, semi):
        pltpu.make_async_copy(src_hbm.at[wid, s], srcb, semi).wait()
        pltpu.make_async_copy(dst_hbm.at[wid, s], dstb, semi).wait()
        pltpu.make_async_copy(w_hbm.at[wid, s], wb, semi).wait()

    with jax.named_scope("peel"):
        _wait_idx(src0_v, dst0_v, w0_v, 0, sem_i0)
        pltpu.async_copy(src_hbm.at[wid, 1], src1_v, sem_i1)
        pltpu.async_copy(dst_hbm.at[wid, 1], dst1_v, sem_i1)
        pltpu.async_copy(w_hbm.at[wid, 1], w1_v, sem_i1)
        _gather(src0_v, 0, 0)
        _gather(src0_v, 1, 1)
        _wait_gather(0)
        _scale(w0_v, 0, rows0_v)
        _scatter(dst0_v, 0, 0)
        _gather(src0_v, 2, 2)
        _wait_gather(1)
        _scale(w0_v, 1, rows1_v)
        _scatter(dst0_v, 1, 1)
        _wait_scatter(0)
        _gather(src0_v, 3, 0)
        _wait_gather(2)
        _scale(w0_v, 2, rows2_v)
        _scatter(dst0_v, 2, 2)
        _wait_scatter(1)
        _gather(src0_v, 4, 1)

    def _slot(lp, t_last, next_ok, pf_same, pf_next,
              srcb, srcb_n, dstb, wb, b):
        bf = (b + 2) % 3
        _wait_gather(b)
        _scale(wb, lp, rows[b])
        _scatter(dstb, lp, b)
        _wait_scatter(bf)

        if pf_next is None:
            _gather(srcb, pf_same, bf)
        else:
            @pl.when(jnp.logical_not(t_last))
            def _pf_same():
                _gather(srcb, pf_same, bf)

            @pl.when(jnp.logical_and(t_last, next_ok))
            def _pf_next():
                _gather(srcb_n, pf_next, bf)

    def _super(s, t_lo, srcb, dstb, wb, semi, srcb_n, dstb_n, wb_n, semi_n):
        @pl.when(s > 0)
        def _w():
            _wait_idx(srcb, dstb, wb, s, semi)

        @pl.when(s + 1 < NSUPER)
        def _prefetch_idx():
            pltpu.async_copy(src_hbm.at[wid, s + 1], srcb_n, semi_n)
            pltpu.async_copy(dst_hbm.at[wid, s + 1], dstb_n, semi_n)
            pltpu.async_copy(w_hbm.at[wid, s + 1], wb_n, semi_n)

        has_next = s + 1 < NSUPER

        def _triple(t, c):
            lp = 3 * t
            t_last = t == NTRIP - 1
            _slot(lp, t_last, True, lp + 2, None,
                  srcb, srcb_n, dstb, wb, 0)
            _slot(lp + 1, t_last, has_next, lp + 3, 0,
                  srcb, srcb_n, dstb, wb, 1)
            _slot(lp + 2, t_last, has_next, lp + 4, 1,
                  srcb, srcb_n, dstb, wb, 2)
            return c
        lax.fori_loop(t_lo, NTRIP, _triple, 0)

    def _souter(s2, c):
        s = 2 * s2
        _super(s, jnp.where(s2 == 0, 1, 0),
               src0_v, dst0_v, w0_v, sem_i0,
               src1_v, dst1_v, w1_v, sem_i1)
        _super(s + 1, 0,
               src1_v, dst1_v, w1_v, sem_i1,
               src0_v, dst0_v, w0_v, sem_i0)
        return c
    with jax.named_scope("mainloop"):
        lax.fori_loop(0, NSUPER // 2, _souter, 0)
        _wait_scatter(2)
    plsc.subcore_barrier()

    with jax.named_scope("copyout"):
        for r in range(ROWS_PER_TILE // ZCHUNK):
            base = sid * ROWS_PER_TILE + r * ZCHUNK
            pltpu.sync_copy(acc_sh.at[pl.ds(base, ZCHUNK)], rows0_v)
            pltpu.sync_copy(rows0_v, out_hbm.at[cid, pl.ds(base, ZCHUNK)])


_sc_segment = functools.partial(
    pl.kernel,
    mesh=plsc.VectorSubcoreMesh(core_axis_name="c", subcore_axis_name="s"),
    out_type=jax.ShapeDtypeStruct((NC, N_ACC, D), jnp.float32),
    scratch_types=[
        pltpu.VMEM((SUPER, 1, K), jnp.int32),
        pltpu.VMEM((SUPER, 1, K), jnp.int32),
        pltpu.VMEM((SUPER, 1, K), jnp.int32),
        pltpu.VMEM((SUPER, 1, K), jnp.int32),
        pltpu.VMEM((SUPER, 1, K), jnp.float32),
        pltpu.VMEM((SUPER, 1, K), jnp.float32),
        pltpu.VMEM((K, D), jnp.float32),
        pltpu.VMEM((K, D), jnp.float32),
        pltpu.VMEM((K, D), jnp.float32),
        pltpu.VMEM_SHARED((N_ACC, D), jnp.float32),
        pltpu.SemaphoreType.DMA,
        pltpu.SemaphoreType.DMA,
        pltpu.SemaphoreType.DMA,
        pltpu.SemaphoreType.DMA,
        pltpu.SemaphoreType.DMA,
        pltpu.SemaphoreType.DMA,
        pltpu.SemaphoreType.DMA,
        pltpu.SemaphoreType.DMA,
    ],
)(_sc_body)


def _dense_body(p_ref, x_ref, wr_ref, ws_ref, b_ref, o_ref, *, act):
    agg = p_ref[0] + p_ref[1]
    z = jnp.dot(agg, wr_ref[...], preferred_element_type=jnp.float32)
    z = z + jnp.dot(x_ref[...], ws_ref[...], preferred_element_type=jnp.float32)
    z = z + b_ref[...]
    if act:
        z = jnp.where(z > 0, z, jnp.exp(z) - 1.0)
    o_ref[...] = z


def _dense(partials, x, wrT, wsT, b, act):
    R = 1000
    return pl.pallas_call(
        functools.partial(_dense_body, act=act),
        grid=(N // R,),
        in_specs=[
            pl.BlockSpec((NC, R, D), lambda i: (0, i, 0)),
            pl.BlockSpec((R, D), lambda i: (i, 0)),
            pl.BlockSpec((D, D), lambda i: (0, 0)),
            pl.BlockSpec((D, D), lambda i: (0, 0)),
            pl.BlockSpec((1, D), lambda i: (0, 0)),
        ],
        out_specs=pl.BlockSpec((R, D), lambda i: (i, 0)),
        out_shape=jax.ShapeDtypeStruct((N, D), jnp.float32),
    )(partials, x, wrT, wsT, b)


def kernel(x, edge_index, edge_attr, W1r, b1, W1s, W2r, b2, W2s):
    src = edge_index[0].astype(jnp.int32)
    dst = edge_index[1].astype(jnp.int32)
    w = edge_attr.astype(jnp.float32)

    pad = E_PAD - E
    spread = jnp.arange(pad, dtype=jnp.int32) % N
    src = jnp.concatenate([src, spread])
    dst = jnp.concatenate([dst, spread])
    w = jnp.concatenate([w, jnp.zeros((pad,), jnp.float32)])
    srcr = src.reshape(NW, NSUPER, SUPER, 1, K)
    dstr = dst.reshape(NW, NSUPER, SUPER, 1, K)
    wr = w.reshape(NW, NSUPER, SUPER, 1, K)

    w1rT = W1r.T
    w1sT = W1s.T
    w2rT = W2r.T
    w2sT = W2s.T
    b1r = b1.reshape(1, D)
    b2r = b2.reshape(1, D)

    p1 = _sc_segment(x, srcr, dstr, wr)
    h = _dense(p1, x, w1rT, w1sT, b1r, act=True)
    p2 = _sc_segment(h, srcr, dstr, wr)
    out = _dense(p2, h, w2rT, w2sT, b2r, act=False)
    return out


# speedup vs baseline: 1.0648x; 1.0648x over previous
import functools

import jax
import jax.numpy as jnp
from jax import lax
from jax.experimental import pallas as pl
from jax.experimental.pallas import tpu as pltpu
from jax.experimental.pallas import tpu_sc as plsc

N = 10000
E = 320000
D = 128

NC = 2
NS = 16
NW = NC * NS

K = 128
NCHUNK = 80
EPT = NCHUNK * K
E_PAD = NW * EPT

N_ACC = 10240
ROWS_PER_TILE = N_ACC // NS
ZCHUNK = 128

SUPER = 8
NSUPER = NCHUNK // SUPER


def _sc_body(x_hbm, src_hbm, dst_hbm, w_hbm, out_hbm,
             src_v, dst0_v, dst1_v, w0_v, w1_v, rows0_v, rows1_v, acc_sh,
             sem_r0, sem_r1, sem_i0, sem_i1):
    cid = lax.axis_index("c")
    sid = lax.axis_index("s")
    wid = sid * NC + cid

    with jax.named_scope("idx_slab"):
        pltpu.sync_copy(src_hbm.at[wid], src_v)
        pltpu.async_copy(dst_hbm.at[wid, 0], dst0_v, sem_i0)
        pltpu.async_copy(w_hbm.at[wid, 0], w0_v, sem_i0)

    with jax.named_scope("zero_acc"):
        def _zero_body(i, carry):
            for c in range(D // 16):
                rows0_v[i, pl.ds(c * 16, 16)] = jnp.zeros((16,), jnp.float32)
            return carry
        lax.fori_loop(0, ZCHUNK, _zero_body, 0)
        for r in range(ROWS_PER_TILE // ZCHUNK):
            pltpu.sync_copy(rows0_v.at[pl.ds(0, ZCHUNK)],
                            acc_sh.at[pl.ds(sid * ROWS_PER_TILE + r * ZCHUNK,
                                            ZCHUNK)])
        plsc.subcore_barrier()

    pltpu.async_copy(x_hbm.at[src_v.at[0, 0]], rows0_v, sem_r0)
    pltpu.async_copy(x_hbm.at[src_v.at[1, 0]], rows1_v, sem_r1)

    def _scale_scatter(dstb, wb, lp, rows_v):
        def _scale_body(g, c2):
            wv = wb[lp, 0, pl.ds(g * 16, 16)]
            for i in range(16):
                wi = wv[i]
                e = g * 16 + i
                for c in range(D // 16):
                    rows_v[e, pl.ds(c * 16, 16)] = (
                        rows_v[e, pl.ds(c * 16, 16)] * wi)
            return c2
        lax.fori_loop(0, K // 16, _scale_body, 0)
        pltpu.sync_copy(rows_v, acc_sh.at[dstb.at[lp, 0]], add=True)

    def _super(s, dstb, wb, semi, dstb_n, wb_n, semi_n):
        pltpu.make_async_copy(dst_hbm.at[wid, s], dstb, semi).wait()
        pltpu.make_async_copy(w_hbm.at[wid, s], wb, semi).wait()
        sn = s + 1

        @pl.when(sn < NSUPER)
        def _prefetch_idx():
            pltpu.async_copy(dst_hbm.at[wid, sn], dstb_n, semi_n)
            pltpu.async_copy(w_hbm.at[wid, sn], wb_n, semi_n)

        def _pair(p2, c):
            lp = 2 * p2
            j = s * SUPER + lp
            pltpu.make_async_copy(
                x_hbm.at[src_v.at[j, 0]], rows0_v, sem_r0).wait()
            _scale_scatter(dstb, wb, lp, rows0_v)

            @pl.when(j + 2 < NCHUNK)
            def _pf0():
                pltpu.async_copy(
                    x_hbm.at[src_v.at[j + 2, 0]], rows0_v, sem_r0)
            pltpu.make_async_copy(
                x_hbm.at[src_v.at[j + 1, 0]], rows1_v, sem_r1).wait()
            _scale_scatter(dstb, wb, lp + 1, rows1_v)

            @pl.when(j + 3 < NCHUNK)
            def _pf1():
                pltpu.async_copy(
                    x_hbm.at[src_v.at[j + 3, 0]], rows1_v, sem_r1)
            return c
        lax.fori_loop(0, SUPER // 2, _pair, 0)

    def _souter(s2, c):
        s = 2 * s2
        _super(s, dst0_v, w0_v, sem_i0, dst1_v, w1_v, sem_i1)
        _super(s + 1, dst1_v, w1_v, sem_i1, dst0_v, w0_v, sem_i0)
        return c
    with jax.named_scope("mainloop"):
        lax.fori_loop(0, NSUPER // 2, _souter, 0)

    plsc.subcore_barrier()

    with jax.named_scope("copyout"):
        bufs = (rows0_v, rows1_v)
        sems = (sem_r0, sem_r1)
        nr = ROWS_PER_TILE // ZCHUNK
        for r in range(nr):
            base = sid * ROWS_PER_TILE + r * ZCHUNK
            buf = bufs[r % 2].at[pl.ds(0, ZCHUNK)]
            sem = sems[r % 2]
            if r >= 2:
                prev = sid * ROWS_PER_TILE + (r - 2) * ZCHUNK
                pltpu.make_async_copy(
                    buf, out_hbm.at[cid, pl.ds(prev, ZCHUNK)], sem).wait()
            pltpu.sync_copy(acc_sh.at[pl.ds(base, ZCHUNK)], buf)
            pltpu.async_copy(buf, out_hbm.at[cid, pl.ds(base, ZCHUNK)], sem)
        for r in range(nr - 2, nr):
            base = sid * ROWS_PER_TILE + r * ZCHUNK
            pltpu.make_async_copy(
                bufs[r % 2].at[pl.ds(0, ZCHUNK)],
                out_hbm.at[cid, pl.ds(base, ZCHUNK)], sems[r % 2]).wait()


_sc_segment = functools.partial(
    pl.kernel,
    mesh=plsc.VectorSubcoreMesh(core_axis_name="c", subcore_axis_name="s"),
    out_type=jax.ShapeDtypeStruct((NC, N_ACC, D), jnp.float32),
    scratch_types=[
        pltpu.VMEM((NCHUNK, 1, K), jnp.int32),
        pltpu.VMEM((SUPER, 1, K), jnp.int32),
        pltpu.VMEM((SUPER, 1, K), jnp.int32),
        pltpu.VMEM((SUPER, 1, K), jnp.float32),
        pltpu.VMEM((SUPER, 1, K), jnp.float32),
        pltpu.VMEM((K, D), jnp.float32),
        pltpu.VMEM((K, D), jnp.float32),
        pltpu.VMEM_SHARED((N_ACC, D), jnp.float32),
        pltpu.SemaphoreType.DMA,
        pltpu.SemaphoreType.DMA,
        pltpu.SemaphoreType.DMA,
        pltpu.SemaphoreType.DMA,
    ],
)(_sc_body)


def _dense_body(p_ref, x_ref, wr_ref, ws_ref, b_ref, o_ref, *, act):
    agg = p_ref[0] + p_ref[1]
    z = jnp.dot(agg, wr_ref[...], preferred_element_type=jnp.float32)
    z = z + jnp.dot(x_ref[...], ws_ref[...], preferred_element_type=jnp.float32)
    z = z + b_ref[...]
    if act:
        z = jnp.where(z > 0, z, jnp.exp(z) - 1.0)
    o_ref[...] = z


def _dense(partials, x, wrT, wsT, b, act):
    R = 1000
    return pl.pallas_call(
        functools.partial(_dense_body, act=act),
        grid=(N // R,),
        in_specs=[
            pl.BlockSpec((NC, R, D), lambda i: (0, i, 0)),
            pl.BlockSpec((R, D), lambda i: (i, 0)),
            pl.BlockSpec((D, D), lambda i: (0, 0)),
            pl.BlockSpec((D, D), lambda i: (0, 0)),
            pl.BlockSpec((1, D), lambda i: (0, 0)),
        ],
        out_specs=pl.BlockSpec((R, D), lambda i: (i, 0)),
        out_shape=jax.ShapeDtypeStruct((N, D), jnp.float32),
    )(partials, x, wrT, wsT, b)


def kernel(x, edge_index, edge_attr, W1r, b1, W1s, W2r, b2, W2s):
    src = edge_index[0].astype(jnp.int32)
    dst = edge_index[1].astype(jnp.int32)
    w = edge_attr.astype(jnp.float32)

    pad = E_PAD - E
    spread = jnp.arange(pad, dtype=jnp.int32) % N
    src = jnp.concatenate([src, spread])
    dst = jnp.concatenate([dst, spread])
    w = jnp.concatenate([w, jnp.zeros((pad,), jnp.float32)])
    srcr = src.reshape(NW, NCHUNK, 1, K)
    dstr = dst.reshape(NW, NSUPER, SUPER, 1, K)
    wr = w.reshape(NW, NSUPER, SUPER, 1, K)

    w1rT = W1r.T
    w1sT = W1s.T
    w2rT = W2r.T
    w2sT = W2s.T
    b1r = b1.reshape(1, D)
    b2r = b2.reshape(1, D)

    p1 = _sc_segment(x, srcr, dstr, wr)
    h = _dense(p1, x, w1rT, w1sT, b1r, act=True)
    p2 = _sc_segment(h, srcr, dstr, wr)
    out = _dense(p2, h, w2rT, w2sT, b2r, act=False)
    return out


# speedup vs baseline: 1.0814x; 1.0156x over previous
import functools

import jax
import jax.numpy as jnp
from jax import lax
from jax.experimental import pallas as pl
from jax.experimental.pallas import tpu as pltpu
from jax.experimental.pallas import tpu_sc as plsc

N = 10000
E = 320000
D = 128

NC = 2
NS = 16
NW = NC * NS

K = 128
NCHUNK = 80
EPT = NCHUNK * K
E_PAD = NW * EPT

N_ACC = 10240
ROWS_PER_TILE = N_ACC // NS
ZCHUNK = 128

SUPER = 8
NSUPER = NCHUNK // SUPER


def _sc_body(x_hbm, src_hbm, dst_hbm, w_hbm, out_hbm,
             src_v, dst0_v, dst1_v, w0_v, w1_v, rows0_v, rows1_v, acc_sh,
             sem_r0, sem_r1, sem_i0, sem_i1):
    cid = lax.axis_index("c")
    sid = lax.axis_index("s")
    wid = sid * NC + cid

    with jax.named_scope("idx_slab"):
        pltpu.sync_copy(src_hbm.at[wid], src_v)
        pltpu.async_copy(dst_hbm.at[wid, 0], dst0_v, sem_i0)
        pltpu.async_copy(w_hbm.at[wid, 0], w0_v, sem_i0)

    with jax.named_scope("zero_acc"):
        def _zero_body(i, carry):
            for c in range(D // 16):
                rows0_v[i, pl.ds(c * 16, 16)] = jnp.zeros((16,), jnp.float32)
            return carry
        lax.fori_loop(0, ZCHUNK, _zero_body, 0)
        for r in range(ROWS_PER_TILE // ZCHUNK):
            pltpu.sync_copy(rows0_v.at[pl.ds(0, ZCHUNK)],
                            acc_sh.at[pl.ds(sid * ROWS_PER_TILE + r * ZCHUNK,
                                            ZCHUNK)])
        plsc.subcore_barrier()

    pltpu.async_copy(x_hbm.at[src_v.at[0, 0]], rows0_v, sem_r0)
    pltpu.async_copy(x_hbm.at[src_v.at[1, 0]], rows1_v, sem_r1)

    def _scale_scatter(dstb, wb, lp, rows_v):
        def _scale_body(g, c2):
            wv = wb[lp, 0, pl.ds(g * 16, 16)]
            for i in range(16):
                wi = wv[i]
                e = g * 16 + i
                for c in range(D // 16):
                    rows_v[e, pl.ds(c * 16, 16)] = (
                        rows_v[e, pl.ds(c * 16, 16)] * wi)
            return c2
        lax.fori_loop(0, K // 16, _scale_body, 0)
        pltpu.sync_copy(rows_v, acc_sh.at[dstb.at[lp, 0]], add=True)

    def _super(s, dstb, wb, semi, dstb_n, wb_n, semi_n):
        pltpu.make_async_copy(dst_hbm.at[wid, s], dstb, semi).wait()
        pltpu.make_async_copy(w_hbm.at[wid, s], wb, semi).wait()
        sn = s + 1

        @pl.when(sn < NSUPER)
        def _prefetch_idx():
            pltpu.async_copy(dst_hbm.at[wid, sn], dstb_n, semi_n)
            pltpu.async_copy(w_hbm.at[wid, sn], wb_n, semi_n)

        def _pair(p2, c):
            lp = 2 * p2
            j = s * SUPER + lp
            pltpu.make_async_copy(
                x_hbm.at[src_v.at[j, 0]], rows0_v, sem_r0).wait()
            _scale_scatter(dstb, wb, lp, rows0_v)

            @pl.when(j + 2 < NCHUNK)
            def _pf0():
                pltpu.async_copy(
                    x_hbm.at[src_v.at[j + 2, 0]], rows0_v, sem_r0)
            pltpu.make_async_copy(
                x_hbm.at[src_v.at[j + 1, 0]], rows1_v, sem_r1).wait()
            _scale_scatter(dstb, wb, lp + 1, rows1_v)

            @pl.when(j + 3 < NCHUNK)
            def _pf1():
                pltpu.async_copy(
                    x_hbm.at[src_v.at[j + 3, 0]], rows1_v, sem_r1)
            return c
        lax.fori_loop(0, SUPER // 2, _pair, 0)

    def _souter(s2, c):
        s = 2 * s2
        _super(s, dst0_v, w0_v, sem_i0, dst1_v, w1_v, sem_i1)
        _super(s + 1, dst1_v, w1_v, sem_i1, dst0_v, w0_v, sem_i0)
        return c
    with jax.named_scope("mainloop"):
        lax.fori_loop(0, NSUPER // 2, _souter, 0)

    plsc.subcore_barrier()

    with jax.named_scope("copyout"):
        bufs = (rows0_v, rows1_v)
        sems = (sem_r0, sem_r1)
        nr = ROWS_PER_TILE // ZCHUNK
        for r in range(nr):
            base = sid * ROWS_PER_TILE + r * ZCHUNK
            buf = bufs[r % 2].at[pl.ds(0, ZCHUNK)]
            sem = sems[r % 2]
            if r >= 2:
                prev = sid * ROWS_PER_TILE + (r - 2) * ZCHUNK
                pltpu.make_async_copy(
                    buf, out_hbm.at[cid, pl.ds(prev, ZCHUNK)], sem).wait()
            pltpu.sync_copy(acc_sh.at[pl.ds(base, ZCHUNK)], buf)
            pltpu.async_copy(buf, out_hbm.at[cid, pl.ds(base, ZCHUNK)], sem)
        for r in range(nr - 2, nr):
            base = sid * ROWS_PER_TILE + r * ZCHUNK
            pltpu.make_async_copy(
                bufs[r % 2].at[pl.ds(0, ZCHUNK)],
                out_hbm.at[cid, pl.ds(base, ZCHUNK)], sems[r % 2]).wait()


_sc_segment = functools.partial(
    pl.kernel,
    mesh=plsc.VectorSubcoreMesh(core_axis_name="c", subcore_axis_name="s"),
    out_type=jax.ShapeDtypeStruct((NC, N_ACC, D), jnp.float32),
    scratch_types=[
        pltpu.VMEM((NCHUNK, 1, K), jnp.int32),
        pltpu.VMEM((SUPER, 1, K), jnp.int32),
        pltpu.VMEM((SUPER, 1, K), jnp.int32),
        pltpu.VMEM((SUPER, 1, K), jnp.float32),
        pltpu.VMEM((SUPER, 1, K), jnp.float32),
        pltpu.VMEM((K, D), jnp.float32),
        pltpu.VMEM((K, D), jnp.float32),
        pltpu.VMEM_SHARED((N_ACC, D), jnp.float32),
        pltpu.SemaphoreType.DMA,
        pltpu.SemaphoreType.DMA,
        pltpu.SemaphoreType.DMA,
        pltpu.SemaphoreType.DMA,
    ],
)(_sc_body)


def _dense_body(p_ref, x_ref, wr_ref, ws_ref, b_ref, o_ref, *, act):
    agg = p_ref[0] + p_ref[1]
    z = jnp.dot(agg, wr_ref[...], preferred_element_type=jnp.float32)
    z = z + jnp.dot(x_ref[...], ws_ref[...], preferred_element_type=jnp.float32)
    z = z + b_ref[...]
    if act:
        z = jnp.where(z > 0, z, jnp.exp(z) - 1.0)
    o_ref[...] = z


def _dense(partials, x, wrT, wsT, b, act):
    R = 2000
    return pl.pallas_call(
        functools.partial(_dense_body, act=act),
        grid=(N // R,),
        in_specs=[
            pl.BlockSpec((NC, R, D), lambda i: (0, i, 0)),
            pl.BlockSpec((R, D), lambda i: (i, 0)),
            pl.BlockSpec((D, D), lambda i: (0, 0)),
            pl.BlockSpec((D, D), lambda i: (0, 0)),
            pl.BlockSpec((1, D), lambda i: (0, 0)),
        ],
        out_specs=pl.BlockSpec((R, D), lambda i: (i, 0)),
        out_shape=jax.ShapeDtypeStruct((N, D), jnp.float32),
    )(partials, x, wrT, wsT, b)


def kernel(x, edge_index, edge_attr, W1r, b1, W1s, W2r, b2, W2s):
    src = edge_index[0].astype(jnp.int32)
    dst = edge_index[1].astype(jnp.int32)
    w = edge_attr.astype(jnp.float32)

    pad = E_PAD - E
    spread = jnp.arange(pad, dtype=jnp.int32) % N
    src = jnp.concatenate([src, spread])
    dst = jnp.concatenate([dst, spread])
    w = jnp.concatenate([w, jnp.zeros((pad,), jnp.float32)])
    srcr = src.reshape(NW, NCHUNK, 1, K)
    dstr = dst.reshape(NW, NSUPER, SUPER, 1, K)
    wr = w.reshape(NW, NSUPER, SUPER, 1, K)

    w1rT = W1r.T
    w1sT = W1s.T
    w2rT = W2r.T
    w2sT = W2s.T
    b1r = b1.reshape(1, D)
    b2r = b2.reshape(1, D)

    p1 = _sc_segment(x, srcr, dstr, wr)
    h = _dense(p1, x, w1rT, w1sT, b1r, act=True)
    p2 = _sc_segment(h, srcr, dstr, wr)
    out = _dense(p2, h, w2rT, w2sT, b2r, act=False)
    return out
